# Initial kernel scaffold; baseline (speedup 1.0000x reference)
import jax
import jax.numpy as jnp
from jax.experimental import pallas as pl


def kernel(atoms, bonds, edges, W1, b1, W2, b2, Wo, bo):
    raise NotImplementedError("write your pallas kernel here")


# speedup vs baseline: 39.4370x; 39.4370x over previous
import jax
import jax.numpy as jnp
from jax import lax
from jax.experimental import pallas as pl
from jax.experimental.pallas import tpu as pltpu

_N = 128
_D = 6
_ND = 7
_BF = 13
_AF = 128
_H = 1024


def _mol_kernel(atoms_ref, bonds_ref, edges_ref,
                w1a_ref, w1b_ref, b1_ref,
                w2a_ref, w2b_ref, b2_ref,
                woa_ref, wob_ref, bo_ref,
                out_ref):
    f32 = jnp.float32
    x = atoms_ref[0]
    b78 = bonds_ref[0]
    e = edges_ref[0]

    si = lax.broadcasted_iota(jnp.int32, (_D * _BF, _BF), 0)
    sj = lax.broadcasted_iota(jnp.int32, (_D * _BF, _BF), 1)
    sel = (si % _BF == sj).astype(f32)
    sb = jnp.dot(b78, sel, preferred_element_type=f32)

    colids = lax.broadcasted_iota(jnp.int32, (_N, _N), 1)
    rowids = lax.broadcasted_iota(jnp.int32, (_N, _N), 0)
    eye = (colids == rowids).astype(f32)

    A = eye
    for d in range(_D):
        A = A + (e[:, d:d + 1] == colids).astype(f32)

    deg = jnp.sum((e != -1).astype(f32), axis=1, keepdims=True)

    def conv(xin, wa, wbflat, bflat):
        s_atoms = jnp.dot(A, xin, preferred_element_type=f32)
        z_all = (jnp.dot(s_atoms, wa, preferred_element_type=f32)
                 + jnp.dot(sb, wbflat, preferred_element_type=f32)
                 + bflat)
        acc = jnp.zeros((_N, 128), f32)
        for d in range(_ND):
            zd = z_all[:, d * 128:(d + 1) * 128]
            acc = acc + jnp.maximum(zd, 0.0) * (deg == d).astype(f32)
        return acc

    def pool(h):
        g = h
        for d in range(_D):
            ed = e[:, d:d + 1]
            onehot = (ed == colids).astype(f32)
            gd = jnp.dot(onehot, h, preferred_element_type=f32)
            gd = jnp.where(ed >= 0, gd, -jnp.inf)
            g = jnp.maximum(g, gd)
        return g

    h1 = conv(x, w1a_ref[...], w1b_ref[...], b1_ref[...])
    p1 = pool(h1)
    h2 = conv(p1, w2a_ref[...], w2b_ref[...], b2_ref[...])
    p2 = pool(h2)

    z = (jnp.dot(p2, woa_ref[...], preferred_element_type=f32)
         + jnp.dot(sb, wob_ref[...], preferred_element_type=f32)
         + bo_ref[...])
    fp = jnp.tanh(z) * (deg != 0).astype(f32)
    out_ref[0] = jnp.sum(fp, axis=0, keepdims=True)


def kernel(atoms, bonds, edges, W1, b1, W2, b2, Wo, bo):
    B = atoms.shape[0]
    b78 = bonds.reshape(B, _N, _D * _BF)

    def split_w(W, b):
        wa = jnp.transpose(W[:, :_AF, :], (1, 0, 2)).reshape(_AF, _ND * 128)
        wb = jnp.transpose(W[:, _AF:, :], (1, 0, 2)).reshape(_BF, _ND * 128)
        return wa, wb, b.reshape(1, _ND * 128)

    w1a, w1b, b1f = split_w(W1, b1)
    w2a, w2b, b2f = split_w(W2, b2)
    woa = Wo[:_AF]
    wob = Wo[_AF:]
    bof = bo.reshape(1, _H)

    const = lambda i: (0, 0)
    return pl.pallas_call(
        _mol_kernel,
        grid=(B,),
        in_specs=[
            pl.BlockSpec((1, _N, _AF), lambda i: (i, 0, 0)),
            pl.BlockSpec((1, _N, _D * _BF), lambda i: (i, 0, 0)),
            pl.BlockSpec((1, _N, _D), lambda i: (i, 0, 0)),
            pl.BlockSpec((_AF, _ND * 128), const),
            pl.BlockSpec((_BF, _ND * 128), const),
            pl.BlockSpec((1, _ND * 128), const),
            pl.BlockSpec((_AF, _ND * 128), const),
            pl.BlockSpec((_BF, _ND * 128), const),
            pl.BlockSpec((1, _ND * 128), const),
            pl.BlockSpec((_AF, _H), const),
            pl.BlockSpec((_BF, _H), const),
            pl.BlockSpec((1, _H), const),
        ],
        out_specs=pl.BlockSpec((1, 1, _H), lambda i: (i, 0, 0)),
        out_shape=jax.ShapeDtypeStruct((B, 1, _H), jnp.float32),
        compiler_params=pltpu.CompilerParams(
            dimension_semantics=("parallel",)),
    )(atoms, b78, edges, w1a, w1b, b1f, w2a, w2b, b2f, woa, wob, bof
      ).reshape(B, _H)
